# Initial kernel scaffold; baseline (speedup 1.0000x reference)
import jax
import jax.numpy as jnp
from jax.experimental import pallas as pl


def kernel(queries, db_weight, db_label, WQ, WK, WV, WO, bQ, bK, bV, bO, dense_w, dense_b, out_w, out_b):
    raise NotImplementedError("write your pallas kernel here")


# speedup vs baseline: 1.8417x; 1.8417x over previous
import functools

import jax
import jax.numpy as jnp
import numpy as np
from jax import lax
from jax.experimental import pallas as pl
from jax.experimental.pallas import tpu as pltpu

D = 640
N_DB = 100000
Q = 1024
K_TOP = 3
N_LABELS = 12
N_LAYERS = 3
RATIO = 0.2
D_K = D // 8

N_TILE = 2048
N_TILES = (N_DB + N_TILE - 1) // N_TILE
N_PAD = N_TILES * N_TILE
GCOLS = D + 128
ROWS = Q * K_TOP
QBLK = 128
RBLK = QBLK * K_TOP
N_QBLKS = Q // QBLK

_NEG = float(np.finfo(np.float32).min)
_IMAX = np.int32(np.iinfo(np.int32).max)


def _topk_body(q_ref, db_ref, lbl_ref, tv_ref, ti_ref, dbt_ref,
               qn_ref, rv_ref, ri_ref):
    t = pl.program_id(0)

    @pl.when(t == 0)
    def _init():
        qr = q_ref[...]
        qn_ref[...] = qr * lax.rsqrt(
            jnp.sum(qr * qr, axis=1, keepdims=True))
        rv_ref[...] = jnp.full((Q, K_TOP), _NEG, jnp.float32)
        ri_ref[...] = jnp.zeros((Q, K_TOP), jnp.int32)

    blk = db_ref[...]
    sims = jnp.dot(qn_ref[...], blk, preferred_element_type=jnp.float32)

    col = t * N_TILE + lax.broadcasted_iota(jnp.int32, (Q, N_TILE), 1)
    sims = jnp.where(col < N_DB, sims, _NEG)

    tvals, tidxs = [], []
    for _ in range(K_TOP):
        m = jnp.max(sims, axis=1, keepdims=True)
        am = jnp.min(jnp.where(sims == m, col, _IMAX), axis=1, keepdims=True)
        tvals.append(m)
        tidxs.append(am)
        sims = jnp.where(col == am, _NEG, sims)

    cv = jnp.concatenate([rv_ref[...]] + tvals, axis=1)
    ci = jnp.concatenate([ri_ref[...]] + tidxs, axis=1)
    nv, ni = [], []
    for _ in range(K_TOP):
        m = jnp.max(cv, axis=1, keepdims=True)
        am = jnp.min(jnp.where(cv == m, ci, _IMAX), axis=1, keepdims=True)
        nv.append(m)
        ni.append(am)
        cv = jnp.where((cv == m) & (ci == am), _NEG, cv)
    rv_ref[...] = jnp.concatenate(nv, axis=1)
    ri_ref[...] = jnp.concatenate(ni, axis=1)

    blk_t = blk.T
    lblf = lbl_ref[0, 0, :].astype(jnp.float32).reshape(N_TILE, 1)
    pad = jnp.zeros((N_TILE, GCOLS - D - 1), jnp.float32)
    dbt_ref[...] = jnp.concatenate([blk_t, lblf, pad], axis=1)

    @pl.when(t == N_TILES - 1)
    def _emit():
        tv_ref[...] = rv_ref[...]
        ti_ref[...] = ri_ref[...]


def _run_topk(queries, db_weight, db_label):
    lbl = jnp.pad(db_label, (0, N_PAD - N_DB)).reshape(N_TILES, 1, N_TILE)
    return pl.pallas_call(
        _topk_body,
        grid=(N_TILES,),
        in_specs=[
            pl.BlockSpec((Q, D), lambda t: (0, 0)),
            pl.BlockSpec((D, N_TILE), lambda t: (0, t)),
            pl.BlockSpec((1, 1, N_TILE), lambda t: (t, 0, 0)),
        ],
        out_specs=[
            pl.BlockSpec((Q, K_TOP), lambda t: (0, 0)),
            pl.BlockSpec((Q, K_TOP), lambda t: (0, 0)),
            pl.BlockSpec((N_TILE, GCOLS), lambda t: (t, 0)),
        ],
        out_shape=[
            jax.ShapeDtypeStruct((Q, K_TOP), jnp.float32),
            jax.ShapeDtypeStruct((Q, K_TOP), jnp.int32),
            jax.ShapeDtypeStruct((N_PAD, GCOLS), jnp.float32),
        ],
        scratch_shapes=[
            pltpu.VMEM((Q, D), jnp.float32),
            pltpu.VMEM((Q, K_TOP), jnp.float32),
            pltpu.VMEM((Q, K_TOP), jnp.int32),
        ],
        compiler_params=pltpu.CompilerParams(
            dimension_semantics=("arbitrary",)),
    )(queries, db_weight, lbl)


def _sc_gather(table, idx_flat):
    from jax.experimental.pallas import tpu_sc as plsc

    info = plsc.get_sparse_core_info()
    nc, ns = info.num_cores, info.num_subcores
    nw = nc * ns
    bpw = ROWS // nw
    mesh = plsc.VectorSubcoreMesh(core_axis_name="c", subcore_axis_name="s")

    @functools.partial(
        pl.kernel, mesh=mesh,
        out_type=jax.ShapeDtypeStruct((ROWS, GCOLS), jnp.float32),
        scratch_types=[
            pltpu.VMEM((bpw,), jnp.int32),
            pltpu.VMEM((bpw, GCOLS), jnp.float32),
            pltpu.SemaphoreType.DMA,
        ],
    )
    def k(table_hbm, idx_hbm, out_hbm, idx_v, rows_v, sem):
        wid = lax.axis_index("s") * nc + lax.axis_index("c")
        base = wid * bpw
        pltpu.sync_copy(idx_hbm.at[pl.ds(base, bpw)], idx_v)
        pltpu.async_copy(table_hbm.at[idx_v], rows_v, sem).wait()
        pltpu.sync_copy(rows_v, out_hbm.at[pl.ds(base, bpw)])

    return k(table, idx_flat)


def _attn_body(seq_ref, q_ref, sc_ref,
               wq_ref, wk_ref, wv_ref, wo_ref,
               bq_ref, bk_ref, bv_ref, bo_ref,
               dw_ref, db_ref, ow_ref, ob_ref, out_ref):
    f32 = jnp.float32
    seqs = seq_ref[:, :D]
    lbl = seq_ref[:, D:D + 1].astype(jnp.int32)

    r_over_3 = lax.broadcasted_iota(jnp.int32, (RBLK, QBLK), 0) // K_TOP
    s3 = (r_over_3 == lax.broadcasted_iota(jnp.int32, (RBLK, QBLK), 1))
    s3f = s3.astype(f32)
    hx = jnp.dot(s3f, q_ref[...], preferred_element_type=f32)

    sc3 = jnp.dot(s3f, sc_ref[...], preferred_element_type=f32)
    rmod = lax.broadcasted_iota(jnp.int32, (RBLK, K_TOP), 0) % K_TOP
    kcol = lax.broadcasted_iota(jnp.int32, (RBLK, K_TOP), 1)
    score = jnp.sum(jnp.where(rmod == kcol, sc3, 0.0), axis=1, keepdims=True)

    dcol = lax.broadcasted_iota(jnp.int32, (RBLK, D), 1)
    hc = jnp.where(dcol == lbl, score, 0.0)
    hr = seqs

    inv_sqrt_dk = 1.0 / np.sqrt(D_K)
    for i in range(N_LAYERS):
        wqi, wki = wq_ref[i], wk_ref[i]
        wvi, woi = wv_ref[i], wo_ref[i]
        bqi = bq_ref[i:i + 1, :]
        bki = bk_ref[i:i + 1, :]
        bvi = bv_ref[i:i + 1, :]
        boi = bo_ref[i:i + 1, :]
        qs = [jnp.dot(h, wqi, preferred_element_type=f32) + bqi
              for h in (hc, hx, hr)]
        ks = [jnp.dot(h, wki, preferred_element_type=f32) + bki
              for h in (hc, hx, hr)]
        vs = [jnp.dot(h, wvi, preferred_element_type=f32) + bvi
              for h in (hc, hx, hr)]
        outs = []
        for a in range(3):
            att = [jnp.sum(qs[a] * ks[b], axis=1, keepdims=True) * inv_sqrt_dk
                   for b in range(3)]
            m = jnp.maximum(jnp.maximum(att[0], att[1]), att[2])
            e = [jnp.exp(x - m) for x in att]
            denom = e[0] + e[1] + e[2]
            na = (e[0] * vs[0] + e[1] * vs[1] + e[2] * vs[2]) / denom
            outs.append(jnp.dot(na, woi, preferred_element_type=f32) + boi)
        hc, hx, hr = outs

    x = jnp.tanh(jnp.dot(hc, dw_ref[...], preferred_element_type=f32)
                 + db_ref[...])
    x = jnp.dot(x, ow_ref[...], preferred_element_type=f32) + ob_ref[...]

    lcol = lax.broadcasted_iota(jnp.int32, (RBLK, N_LABELS), 1)
    oh = (lcol == lbl).astype(f32)
    comb = x * (1.0 - RATIO) + oh * RATIO

    qrow = lax.broadcasted_iota(jnp.int32, (QBLK, RBLK), 0)
    rcol = lax.broadcasted_iota(jnp.int32, (QBLK, RBLK), 1) // K_TOP
    savg = jnp.where(qrow == rcol, 1.0 / K_TOP, 0.0)
    out_ref[...] = jnp.dot(savg, comb, preferred_element_type=f32)


def _run_attn(seqs, queries, top_scores,
              WQ, WK, WV, WO, bQ, bK, bV, bO,
              dense_w, dense_b, out_w, out_b):
    whole3 = pl.BlockSpec((N_LAYERS, D, D), lambda g: (0, 0, 0))
    wholeb = pl.BlockSpec((N_LAYERS, D), lambda g: (0, 0))
    return pl.pallas_call(
        _attn_body,
        grid=(N_QBLKS,),
        in_specs=[
            pl.BlockSpec((RBLK, GCOLS), lambda g: (g, 0)),
            pl.BlockSpec((QBLK, D), lambda g: (g, 0)),
            pl.BlockSpec((QBLK, K_TOP), lambda g: (g, 0)),
            whole3, whole3, whole3, whole3,
            wholeb, wholeb, wholeb, wholeb,
            pl.BlockSpec((D, D), lambda g: (0, 0)),
            pl.BlockSpec((1, D), lambda g: (0, 0)),
            pl.BlockSpec((D, N_LABELS), lambda g: (0, 0)),
            pl.BlockSpec((1, N_LABELS), lambda g: (0, 0)),
        ],
        out_specs=pl.BlockSpec((QBLK, N_LABELS), lambda g: (g, 0)),
        out_shape=jax.ShapeDtypeStruct((Q, N_LABELS), jnp.float32),
        compiler_params=pltpu.CompilerParams(
            dimension_semantics=("arbitrary",)),
    )(seqs, queries, top_scores, WQ, WK, WV, WO, bQ, bK, bV, bO,
      dense_w, dense_b.reshape(1, D), out_w, out_b.reshape(1, N_LABELS))


def kernel(queries, db_weight, db_label, WQ, WK, WV, WO,
           bQ, bK, bV, bO, dense_w, dense_b, out_w, out_b):
    top_v, top_i, table = _run_topk(queries, db_weight, db_label)
    seqs = _sc_gather(table, top_i.reshape(-1))
    return _run_attn(seqs, queries, top_v,
                     WQ, WK, WV, WO, bQ, bK, bV, bO,
                     dense_w, dense_b, out_w, out_b)


# speedup vs baseline: 1.9302x; 1.0481x over previous
import functools

import jax
import jax.numpy as jnp
import numpy as np
from jax import lax
from jax.experimental import pallas as pl
from jax.experimental.pallas import tpu as pltpu

D = 640
N_DB = 100000
Q = 1024
K_TOP = 3
N_LABELS = 12
N_LAYERS = 3
RATIO = 0.2
D_K = D // 8

N_TILE = 2048
N_TILES = (N_DB + N_TILE - 1) // N_TILE
N_PAD = N_TILES * N_TILE
GCOLS = D + 128
ROWS = Q * K_TOP
QBLK = 128
RBLK = QBLK * K_TOP
N_QBLKS = Q // QBLK

_NEG = float(np.finfo(np.float32).min)
_IMAX = np.int32(np.iinfo(np.int32).max)


def _topk_body(q_ref, db_ref, lbl_ref, tv_ref, ti_ref, dbt_ref,
               qn_ref, rv_ref, ri_ref):
    t = pl.program_id(0)

    @pl.when(t == 0)
    def _init():
        qr = q_ref[...]
        qn_ref[...] = qr * lax.rsqrt(
            jnp.sum(qr * qr, axis=1, keepdims=True))
        rv_ref[...] = jnp.full((Q, K_TOP), _NEG, jnp.float32)
        ri_ref[...] = jnp.zeros((Q, K_TOP), jnp.int32)

    blk = db_ref[...]
    sims = jnp.dot(qn_ref[...], blk, preferred_element_type=jnp.float32)

    col = t * N_TILE + lax.broadcasted_iota(jnp.int32, (Q, N_TILE), 1)
    sims = jnp.where(col < N_DB, sims, _NEG)

    tvals, tidxs = [], []
    for _ in range(K_TOP):
        m = jnp.max(sims, axis=1, keepdims=True)
        am = jnp.min(jnp.where(sims == m, col, _IMAX), axis=1, keepdims=True)
        tvals.append(m)
        tidxs.append(am)
        sims = jnp.where(col == am, _NEG, sims)

    cv = jnp.concatenate([rv_ref[...]] + tvals, axis=1)
    ci = jnp.concatenate([ri_ref[...]] + tidxs, axis=1)
    nv, ni = [], []
    for _ in range(K_TOP):
        m = jnp.max(cv, axis=1, keepdims=True)
        am = jnp.min(jnp.where(cv == m, ci, _IMAX), axis=1, keepdims=True)
        nv.append(m)
        ni.append(am)
        cv = jnp.where((cv == m) & (ci == am), _NEG, cv)
    rv_ref[...] = jnp.concatenate(nv, axis=1)
    ri_ref[...] = jnp.concatenate(ni, axis=1)

    blk_t = blk.T
    lblf = lbl_ref[0, 0, :].astype(jnp.float32).reshape(N_TILE, 1)
    pad = jnp.zeros((N_TILE, GCOLS - D - 1), jnp.float32)
    dbt_ref[...] = jnp.concatenate([blk_t, lblf, pad], axis=1)

    @pl.when(t == N_TILES - 1)
    def _emit():
        tv_ref[...] = rv_ref[...]
        ti_ref[...] = ri_ref[...]


def _run_topk(queries, db_weight, db_label):
    lbl = jnp.pad(db_label, (0, N_PAD - N_DB)).reshape(N_TILES, 1, N_TILE)
    return pl.pallas_call(
        _topk_body,
        grid=(N_TILES,),
        in_specs=[
            pl.BlockSpec((Q, D), lambda t: (0, 0)),
            pl.BlockSpec((D, N_TILE), lambda t: (0, t)),
            pl.BlockSpec((1, 1, N_TILE), lambda t: (t, 0, 0)),
        ],
        out_specs=[
            pl.BlockSpec((Q, K_TOP), lambda t: (0, 0)),
            pl.BlockSpec((Q, K_TOP), lambda t: (0, 0)),
            pl.BlockSpec((N_TILE, GCOLS), lambda t: (t, 0)),
        ],
        out_shape=[
            jax.ShapeDtypeStruct((Q, K_TOP), jnp.float32),
            jax.ShapeDtypeStruct((Q, K_TOP), jnp.int32),
            jax.ShapeDtypeStruct((N_PAD, GCOLS), jnp.float32),
        ],
        scratch_shapes=[
            pltpu.VMEM((Q, D), jnp.float32),
            pltpu.VMEM((Q, K_TOP), jnp.float32),
            pltpu.VMEM((Q, K_TOP), jnp.int32),
        ],
        compiler_params=pltpu.CompilerParams(
            dimension_semantics=("arbitrary",)),
    )(queries, db_weight, lbl)


def _sc_gather(table, idx_flat):
    from jax.experimental.pallas import tpu_sc as plsc

    info = plsc.get_sparse_core_info()
    nc, ns = info.num_cores, info.num_subcores
    nw = nc * ns
    bpw = ROWS // nw
    mesh = plsc.VectorSubcoreMesh(core_axis_name="c", subcore_axis_name="s")

    @functools.partial(
        pl.kernel, mesh=mesh,
        out_type=jax.ShapeDtypeStruct((ROWS, GCOLS), jnp.float32),
        scratch_types=[
            pltpu.VMEM((bpw,), jnp.int32),
            pltpu.VMEM((bpw, GCOLS), jnp.float32),
            pltpu.SemaphoreType.DMA,
        ],
    )
    def k(table_hbm, idx_hbm, out_hbm, idx_v, rows_v, sem):
        wid = lax.axis_index("s") * nc + lax.axis_index("c")
        base = wid * bpw
        pltpu.sync_copy(idx_hbm.at[pl.ds(base, bpw)], idx_v)
        pltpu.async_copy(table_hbm.at[idx_v], rows_v, sem).wait()
        pltpu.sync_copy(rows_v, out_hbm.at[pl.ds(base, bpw)])

    return k(table, idx_flat)


def _prep_body(wq_ref, wk_ref, wv_ref, wo_ref,
               bq_ref, bk_ref, bv_ref, bo_ref,
               m_ref, vo_ref, sm_ref):
    f32 = jnp.float32
    for i in range(N_LAYERS):
        wq, wk = wq_ref[i], wk_ref[i]
        wv, wo = wv_ref[i], wo_ref[i]
        bq = bq_ref[i:i + 1, :]
        bk = bk_ref[i:i + 1, :]
        bv = bv_ref[i:i + 1, :]
        bo = bo_ref[i:i + 1, :]
        m_ref[i] = lax.dot_general(wq, wk, (((1,), (1,)), ((), ())),
                                   preferred_element_type=f32)
        vo = jnp.dot(wv, wo, preferred_element_type=f32)
        vo_ref[i] = vo
        kv = lax.dot_general(bq, wk, (((1,), (1,)), ((), ())),
                             preferred_element_type=f32)
        qv = lax.dot_general(bk, wq, (((1,), (1,)), ((), ())),
                             preferred_element_type=f32)
        bvo = jnp.dot(bv, wo, preferred_element_type=f32) + bo
        c = jnp.sum(bq * bk) * jnp.ones((1, D), f32)
        z = jnp.zeros((4, D), f32)
        sm_ref[i] = jnp.concatenate([kv, qv, bvo, c, z], axis=0)


def _run_prep(WQ, WK, WV, WO, bQ, bK, bV, bO):
    whole3 = pl.BlockSpec((N_LAYERS, D, D), lambda: (0, 0, 0))
    wholeb = pl.BlockSpec((N_LAYERS, D), lambda: (0, 0))
    return pl.pallas_call(
        _prep_body,
        in_specs=[whole3, whole3, whole3, whole3,
                  wholeb, wholeb, wholeb, wholeb],
        out_specs=[whole3, whole3,
                   pl.BlockSpec((N_LAYERS, 8, D), lambda: (0, 0, 0))],
        out_shape=[
            jax.ShapeDtypeStruct((N_LAYERS, D, D), jnp.float32),
            jax.ShapeDtypeStruct((N_LAYERS, D, D), jnp.float32),
            jax.ShapeDtypeStruct((N_LAYERS, 8, D), jnp.float32),
        ],
    )(WQ, WK, WV, WO, bQ, bK, bV, bO)


def _attn_body(seq_ref, q_ref, sc_ref, m_ref, vo_ref, sm_ref,
               dw_ref, db_ref, ow_ref, ob_ref, out_ref):
    f32 = jnp.float32
    seqs = seq_ref[:, :D]
    lbl = seq_ref[:, D:D + 1].astype(jnp.int32)

    r_over_3 = lax.broadcasted_iota(jnp.int32, (RBLK, QBLK), 0) // K_TOP
    s3f = (r_over_3 == lax.broadcasted_iota(
        jnp.int32, (RBLK, QBLK), 1)).astype(f32)
    qblk = q_ref[...]
    hx = jnp.dot(s3f, qblk, preferred_element_type=f32)

    sc3 = jnp.dot(s3f, sc_ref[...], preferred_element_type=f32)
    rmod = lax.broadcasted_iota(jnp.int32, (RBLK, K_TOP), 0) % K_TOP
    kcol = lax.broadcasted_iota(jnp.int32, (RBLK, K_TOP), 1)
    score = jnp.sum(jnp.where(rmod == kcol, sc3, 0.0), axis=1, keepdims=True)

    ccol = lax.broadcasted_iota(jnp.int32, (RBLK, 16), 1)
    ohs = jnp.where(ccol == lbl, score, 0.0)

    inv_sqrt_dk = 1.0 / np.sqrt(D_K)

    def layer_tail(ps, hdots, ts, us, c, ys, bvo):
        outs = []
        for a in range(3):
            att = [(hdots[a][b] + ts[a] + us[b] + c) * inv_sqrt_dk
                   for b in range(3)]
            m = jnp.maximum(jnp.maximum(att[0], att[1]), att[2])
            e = [jnp.exp(x - m) for x in att]
            denom = e[0] + e[1] + e[2]
            outs.append((e[0] * ys[0] + e[1] * ys[1] + e[2] * ys[2]) / denom
                        + bvo)
        return outs

    m1, vo1 = m_ref[0], vo_ref[0]
    kv = sm_ref[0, 0:1, :]
    qv = sm_ref[0, 1:2, :]
    bvo = sm_ref[0, 2:3, :]
    c = sm_ref[0, 3:4, 0:1]
    kv16, qv16 = kv[:, :16], qv[:, :16]

    pc = jnp.dot(ohs, m_ref[0, :16, :], preferred_element_type=f32)
    px = jnp.dot(s3f, jnp.dot(qblk, m1, preferred_element_type=f32),
                 preferred_element_type=f32)
    pr = jnp.dot(seqs, m1, preferred_element_type=f32)
    ps = [pc, px, pr]

    yc = jnp.dot(ohs, vo_ref[0, :16, :], preferred_element_type=f32) + 0.0
    yx = jnp.dot(s3f, jnp.dot(qblk, vo1, preferred_element_type=f32),
                 preferred_element_type=f32)
    yr = jnp.dot(seqs, vo1, preferred_element_type=f32)
    ys = [yc, yx, yr]

    ts = [jnp.sum(ohs * qv16, axis=1, keepdims=True),
          jnp.dot(s3f, jnp.sum(qblk * qv, axis=1, keepdims=True),
                  preferred_element_type=f32),
          jnp.sum(seqs * qv, axis=1, keepdims=True)]
    us = [jnp.sum(ohs * kv16, axis=1, keepdims=True),
          jnp.dot(s3f, jnp.sum(qblk * kv, axis=1, keepdims=True),
                  preferred_element_type=f32),
          jnp.sum(seqs * kv, axis=1, keepdims=True)]

    hdots = [[jnp.sum(p[:, :16] * ohs, axis=1, keepdims=True),
              jnp.sum(p * hx, axis=1, keepdims=True),
              jnp.sum(p * seqs, axis=1, keepdims=True)] for p in ps]
    hs = layer_tail(ps, hdots, ts, us, c, ys, bvo)

    for i in range(1, N_LAYERS):
        mi, voi = m_ref[i], vo_ref[i]
        kv = sm_ref[i, 0:1, :]
        qv = sm_ref[i, 1:2, :]
        bvo = sm_ref[i, 2:3, :]
        c = sm_ref[i, 3:4, 0:1]
        ps = [jnp.dot(h, mi, preferred_element_type=f32) for h in hs]
        ys = [jnp.dot(h, voi, preferred_element_type=f32) for h in hs]
        ts = [jnp.sum(h * qv, axis=1, keepdims=True) for h in hs]
        us = [jnp.sum(h * kv, axis=1, keepdims=True) for h in hs]
        hdots = [[jnp.sum(p * h, axis=1, keepdims=True) for h in hs]
                 for p in ps]
        hs = layer_tail(ps, hdots, ts, us, c, ys, bvo)

    x = jnp.tanh(jnp.dot(hs[0], dw_ref[...], preferred_element_type=f32)
                 + db_ref[...])
    x = jnp.dot(x, ow_ref[...], preferred_element_type=f32) + ob_ref[...]

    lcol = lax.broadcasted_iota(jnp.int32, (RBLK, N_LABELS), 1)
    oh = (lcol == lbl).astype(f32)
    comb = x * (1.0 - RATIO) + oh * RATIO

    qrow = lax.broadcasted_iota(jnp.int32, (QBLK, RBLK), 0)
    rcol = lax.broadcasted_iota(jnp.int32, (QBLK, RBLK), 1) // K_TOP
    savg = jnp.where(qrow == rcol, 1.0 / K_TOP, 0.0)
    out_ref[...] = jnp.dot(savg, comb, preferred_element_type=f32)


def _run_attn(seqs, queries, top_scores, M3, VO3, SM3,
              dense_w, dense_b, out_w, out_b):
    whole3 = pl.BlockSpec((N_LAYERS, D, D), lambda g: (0, 0, 0))
    return pl.pallas_call(
        _attn_body,
        grid=(N_QBLKS,),
        in_specs=[
            pl.BlockSpec((RBLK, GCOLS), lambda g: (g, 0)),
            pl.BlockSpec((QBLK, D), lambda g: (g, 0)),
            pl.BlockSpec((QBLK, K_TOP), lambda g: (g, 0)),
            whole3, whole3,
            pl.BlockSpec((N_LAYERS, 8, D), lambda g: (0, 0, 0)),
            pl.BlockSpec((D, D), lambda g: (0, 0)),
            pl.BlockSpec((1, D), lambda g: (0, 0)),
            pl.BlockSpec((D, N_LABELS), lambda g: (0, 0)),
            pl.BlockSpec((1, N_LABELS), lambda g: (0, 0)),
        ],
        out_specs=pl.BlockSpec((QBLK, N_LABELS), lambda g: (g, 0)),
        out_shape=jax.ShapeDtypeStruct((Q, N_LABELS), jnp.float32),
        compiler_params=pltpu.CompilerParams(
            dimension_semantics=("arbitrary",)),
    )(seqs, queries, top_scores, M3, VO3, SM3,
      dense_w, dense_b.reshape(1, D), out_w, out_b.reshape(1, N_LABELS))


def kernel(queries, db_weight, db_label, WQ, WK, WV, WO,
           bQ, bK, bV, bO, dense_w, dense_b, out_w, out_b):
    top_v, top_i, table = _run_topk(queries, db_weight, db_label)
    seqs = _sc_gather(table, top_i.reshape(-1))
    M3, VO3, SM3 = _run_prep(WQ, WK, WV, WO, bQ, bK, bV, bO)
    return _run_attn(seqs, queries, top_v, M3, VO3, SM3,
                     dense_w, dense_b, out_w, out_b)


# speedup vs baseline: 1.9331x; 1.0015x over previous
import functools

import jax
import jax.numpy as jnp
import numpy as np
from jax import lax
from jax.experimental import pallas as pl
from jax.experimental.pallas import tpu as pltpu

D = 640
N_DB = 100000
Q = 1024
K_TOP = 3
N_LABELS = 12
N_LAYERS = 3
RATIO = 0.2
D_K = D // 8

N_TILE = 2048
N_TILES = (N_DB + N_TILE - 1) // N_TILE
N_PAD = N_TILES * N_TILE
GCOLS = D + 128
ROWS = Q * K_TOP
QBLK = 128
RBLK = QBLK * K_TOP
N_QBLKS = Q // QBLK

_NEG = float(np.finfo(np.float32).min)
_IMAX = np.int32(np.iinfo(np.int32).max)


def _topk_body(q_ref, db_ref, lbl_ref, tv_ref, ti_ref, dbt_ref,
               qn_ref, rv_ref, ri_ref):
    t = pl.program_id(0)

    @pl.when(t == 0)
    def _init():
        qr = q_ref[...]
        qn_ref[...] = qr * lax.rsqrt(
            jnp.sum(qr * qr, axis=1, keepdims=True))
        rv_ref[...] = jnp.full((Q, K_TOP), _NEG, jnp.float32)
        ri_ref[...] = jnp.zeros((Q, K_TOP), jnp.int32)

    blk = db_ref[...]
    sims = jnp.dot(qn_ref[...], blk, preferred_element_type=jnp.float32)

    col = t * N_TILE + lax.broadcasted_iota(jnp.int32, (Q, N_TILE), 1)
    colr = t * N_TILE + lax.broadcasted_iota(jnp.int32, (1, N_TILE), 1)
    sims = jnp.where(colr < N_DB, sims, _NEG)

    tvals, tidxs = [], []
    for _ in range(K_TOP):
        m = jnp.max(sims, axis=1, keepdims=True)
        am = jnp.min(jnp.where(sims == m, col, _IMAX), axis=1, keepdims=True)
        tvals.append(m)
        tidxs.append(am)
        sims = jnp.where(col == am, _NEG, sims)

    cv = jnp.concatenate([rv_ref[...]] + tvals, axis=1)
    ci = jnp.concatenate([ri_ref[...]] + tidxs, axis=1)
    nv, ni = [], []
    for _ in range(K_TOP):
        m = jnp.max(cv, axis=1, keepdims=True)
        am = jnp.min(jnp.where(cv == m, ci, _IMAX), axis=1, keepdims=True)
        nv.append(m)
        ni.append(am)
        cv = jnp.where((cv == m) & (ci == am), _NEG, cv)
    rv_ref[...] = jnp.concatenate(nv, axis=1)
    ri_ref[...] = jnp.concatenate(ni, axis=1)

    blk_t = blk.T
    lblf = lbl_ref[0, 0, :].astype(jnp.float32).reshape(N_TILE, 1)
    pad = jnp.zeros((N_TILE, GCOLS - D - 1), jnp.float32)
    dbt_ref[...] = jnp.concatenate([blk_t, lblf, pad], axis=1)

    @pl.when(t == N_TILES - 1)
    def _emit():
        tv_ref[...] = rv_ref[...]
        ti_ref[...] = ri_ref[...]


def _run_topk(queries, db_weight, db_label):
    lbl = jnp.pad(db_label, (0, N_PAD - N_DB)).reshape(N_TILES, 1, N_TILE)
    return pl.pallas_call(
        _topk_body,
        grid=(N_TILES,),
        in_specs=[
            pl.BlockSpec((Q, D), lambda t: (0, 0)),
            pl.BlockSpec((D, N_TILE), lambda t: (0, t)),
            pl.BlockSpec((1, 1, N_TILE), lambda t: (t, 0, 0)),
        ],
        out_specs=[
            pl.BlockSpec((Q, K_TOP), lambda t: (0, 0)),
            pl.BlockSpec((Q, K_TOP), lambda t: (0, 0)),
            pl.BlockSpec((N_TILE, GCOLS), lambda t: (t, 0)),
        ],
        out_shape=[
            jax.ShapeDtypeStruct((Q, K_TOP), jnp.float32),
            jax.ShapeDtypeStruct((Q, K_TOP), jnp.int32),
            jax.ShapeDtypeStruct((N_PAD, GCOLS), jnp.float32),
        ],
        scratch_shapes=[
            pltpu.VMEM((Q, D), jnp.float32),
            pltpu.VMEM((Q, K_TOP), jnp.float32),
            pltpu.VMEM((Q, K_TOP), jnp.int32),
        ],
        compiler_params=pltpu.CompilerParams(
            dimension_semantics=("arbitrary",)),
    )(queries, db_weight, lbl)


def _sc_gather(table, idx_flat):
    from jax.experimental.pallas import tpu_sc as plsc

    info = plsc.get_sparse_core_info()
    nc, ns = info.num_cores, info.num_subcores
    nw = nc * ns
    bpw = ROWS // nw
    mesh = plsc.VectorSubcoreMesh(core_axis_name="c", subcore_axis_name="s")

    @functools.partial(
        pl.kernel, mesh=mesh,
        out_type=jax.ShapeDtypeStruct((ROWS, GCOLS), jnp.float32),
        scratch_types=[
            pltpu.VMEM((bpw,), jnp.int32),
            pltpu.VMEM((bpw, GCOLS), jnp.float32),
            pltpu.SemaphoreType.DMA,
        ],
    )
    def k(table_hbm, idx_hbm, out_hbm, idx_v, rows_v, sem):
        wid = lax.axis_index("s") * nc + lax.axis_index("c")
        base = wid * bpw
        pltpu.sync_copy(idx_hbm.at[pl.ds(base, bpw)], idx_v)
        pltpu.async_copy(table_hbm.at[idx_v], rows_v, sem).wait()
        pltpu.sync_copy(rows_v, out_hbm.at[pl.ds(base, bpw)])

    return k(table, idx_flat)


def _prep_body(wq_ref, wk_ref, wv_ref, wo_ref,
               bq_ref, bk_ref, bv_ref, bo_ref,
               m_ref, vo_ref, sm_ref):
    f32 = jnp.float32
    for i in range(N_LAYERS):
        wq, wk = wq_ref[i], wk_ref[i]
        wv, wo = wv_ref[i], wo_ref[i]
        bq = bq_ref[i:i + 1, :]
        bk = bk_ref[i:i + 1, :]
        bv = bv_ref[i:i + 1, :]
        bo = bo_ref[i:i + 1, :]
        m_ref[i] = lax.dot_general(wq, wk, (((1,), (1,)), ((), ())),
                                   preferred_element_type=f32)
        vo = jnp.dot(wv, wo, preferred_element_type=f32)
        vo_ref[i] = vo
        kv = lax.dot_general(bq, wk, (((1,), (1,)), ((), ())),
                             preferred_element_type=f32)
        qv = lax.dot_general(bk, wq, (((1,), (1,)), ((), ())),
                             preferred_element_type=f32)
        bvo = jnp.dot(bv, wo, preferred_element_type=f32) + bo
        c = jnp.sum(bq * bk) * jnp.ones((1, D), f32)
        z = jnp.zeros((4, D), f32)
        sm_ref[i] = jnp.concatenate([kv, qv, bvo, c, z], axis=0)


def _run_prep(WQ, WK, WV, WO, bQ, bK, bV, bO):
    whole3 = pl.BlockSpec((N_LAYERS, D, D), lambda: (0, 0, 0))
    wholeb = pl.BlockSpec((N_LAYERS, D), lambda: (0, 0))
    return pl.pallas_call(
        _prep_body,
        in_specs=[whole3, whole3, whole3, whole3,
                  wholeb, wholeb, wholeb, wholeb],
        out_specs=[whole3, whole3,
                   pl.BlockSpec((N_LAYERS, 8, D), lambda: (0, 0, 0))],
        out_shape=[
            jax.ShapeDtypeStruct((N_LAYERS, D, D), jnp.float32),
            jax.ShapeDtypeStruct((N_LAYERS, D, D), jnp.float32),
            jax.ShapeDtypeStruct((N_LAYERS, 8, D), jnp.float32),
        ],
    )(WQ, WK, WV, WO, bQ, bK, bV, bO)


def _attn_body(seq_ref, q_ref, sc_ref, m_ref, vo_ref, sm_ref,
               dw_ref, db_ref, ow_ref, ob_ref, out_ref):
    f32 = jnp.float32
    seqs = seq_ref[:, :D]
    lbl = seq_ref[:, D:D + 1].astype(jnp.int32)

    r_over_3 = lax.broadcasted_iota(jnp.int32, (RBLK, QBLK), 0) // K_TOP
    s3f = (r_over_3 == lax.broadcasted_iota(
        jnp.int32, (RBLK, QBLK), 1)).astype(f32)
    qblk = q_ref[...]
    hx = jnp.dot(s3f, qblk, preferred_element_type=f32)

    sc3 = jnp.dot(s3f, sc_ref[...], preferred_element_type=f32)
    rmod = lax.broadcasted_iota(jnp.int32, (RBLK, K_TOP), 0) % K_TOP
    kcol = lax.broadcasted_iota(jnp.int32, (RBLK, K_TOP), 1)
    score = jnp.sum(jnp.where(rmod == kcol, sc3, 0.0), axis=1, keepdims=True)

    ccol = lax.broadcasted_iota(jnp.int32, (RBLK, 16), 1)
    ohs = jnp.where(ccol == lbl, score, 0.0)

    inv_sqrt_dk = 1.0 / np.sqrt(D_K)

    def layer_tail(ps, hdots, ts, us, c, ys, bvo):
        outs = []
        for a in range(3):
            att = [(hdots[a][b] + ts[a] + us[b] + c) * inv_sqrt_dk
                   for b in range(3)]
            m = jnp.maximum(jnp.maximum(att[0], att[1]), att[2])
            e = [jnp.exp(x - m) for x in att]
            denom = e[0] + e[1] + e[2]
            outs.append((e[0] * ys[0] + e[1] * ys[1] + e[2] * ys[2]) / denom
                        + bvo)
        return outs

    m1, vo1 = m_ref[0], vo_ref[0]
    kv = sm_ref[0, 0:1, :]
    qv = sm_ref[0, 1:2, :]
    bvo = sm_ref[0, 2:3, :]
    c = sm_ref[0, 3:4, 0:1]
    kv16, qv16 = kv[:, :16], qv[:, :16]

    pc = jnp.dot(ohs, m_ref[0, :16, :], preferred_element_type=f32)
    px = jnp.dot(s3f, jnp.dot(qblk, m1, preferred_element_type=f32),
                 preferred_element_type=f32)
    pr = jnp.dot(seqs, m1, preferred_element_type=f32)
    ps = [pc, px, pr]

    yc = jnp.dot(ohs, vo_ref[0, :16, :], preferred_element_type=f32) + 0.0
    yx = jnp.dot(s3f, jnp.dot(qblk, vo1, preferred_element_type=f32),
                 preferred_element_type=f32)
    yr = jnp.dot(seqs, vo1, preferred_element_type=f32)
    ys = [yc, yx, yr]

    ts = [jnp.sum(ohs * qv16, axis=1, keepdims=True),
          jnp.dot(s3f, jnp.sum(qblk * qv, axis=1, keepdims=True),
                  preferred_element_type=f32),
          jnp.sum(seqs * qv, axis=1, keepdims=True)]
    us = [jnp.sum(ohs * kv16, axis=1, keepdims=True),
          jnp.dot(s3f, jnp.sum(qblk * kv, axis=1, keepdims=True),
                  preferred_element_type=f32),
          jnp.sum(seqs * kv, axis=1, keepdims=True)]

    hdots = [[jnp.sum(p[:, :16] * ohs, axis=1, keepdims=True),
              jnp.sum(p * hx, axis=1, keepdims=True),
              jnp.sum(p * seqs, axis=1, keepdims=True)] for p in ps]
    hs = layer_tail(ps, hdots, ts, us, c, ys, bvo)

    for i in range(1, N_LAYERS):
        mi, voi = m_ref[i], vo_ref[i]
        kv = sm_ref[i, 0:1, :]
        qv = sm_ref[i, 1:2, :]
        bvo = sm_ref[i, 2:3, :]
        c = sm_ref[i, 3:4, 0:1]
        ps = [jnp.dot(h, mi, preferred_element_type=f32) for h in hs]
        ys = [jnp.dot(h, voi, preferred_element_type=f32) for h in hs]
        ts = [jnp.sum(h * qv, axis=1, keepdims=True) for h in hs]
        us = [jnp.sum(h * kv, axis=1, keepdims=True) for h in hs]
        hdots = [[jnp.sum(p * h, axis=1, keepdims=True) for h in hs]
                 for p in ps]
        hs = layer_tail(ps, hdots, ts, us, c, ys, bvo)

    x = jnp.tanh(jnp.dot(hs[0], dw_ref[...], preferred_element_type=f32)
                 + db_ref[...])
    x = jnp.dot(x, ow_ref[...], preferred_element_type=f32) + ob_ref[...]

    lcol = lax.broadcasted_iota(jnp.int32, (RBLK, N_LABELS), 1)
    oh = (lcol == lbl).astype(f32)
    comb = x * (1.0 - RATIO) + oh * RATIO

    qrow = lax.broadcasted_iota(jnp.int32, (QBLK, RBLK), 0)
    rcol = lax.broadcasted_iota(jnp.int32, (QBLK, RBLK), 1) // K_TOP
    savg = jnp.where(qrow == rcol, 1.0 / K_TOP, 0.0)
    out_ref[...] = jnp.dot(savg, comb, preferred_element_type=f32)


def _run_attn(seqs, queries, top_scores, M3, VO3, SM3,
              dense_w, dense_b, out_w, out_b):
    whole3 = pl.BlockSpec((N_LAYERS, D, D), lambda g: (0, 0, 0))
    return pl.pallas_call(
        _attn_body,
        grid=(N_QBLKS,),
        in_specs=[
            pl.BlockSpec((RBLK, GCOLS), lambda g: (g, 0)),
            pl.BlockSpec((QBLK, D), lambda g: (g, 0)),
            pl.BlockSpec((QBLK, K_TOP), lambda g: (g, 0)),
            whole3, whole3,
            pl.BlockSpec((N_LAYERS, 8, D), lambda g: (0, 0, 0)),
            pl.BlockSpec((D, D), lambda g: (0, 0)),
            pl.BlockSpec((1, D), lambda g: (0, 0)),
            pl.BlockSpec((D, N_LABELS), lambda g: (0, 0)),
            pl.BlockSpec((1, N_LABELS), lambda g: (0, 0)),
        ],
        out_specs=pl.BlockSpec((QBLK, N_LABELS), lambda g: (g, 0)),
        out_shape=jax.ShapeDtypeStruct((Q, N_LABELS), jnp.float32),
        compiler_params=pltpu.CompilerParams(
            dimension_semantics=("arbitrary",)),
    )(seqs, queries, top_scores, M3, VO3, SM3,
      dense_w, dense_b.reshape(1, D), out_w, out_b.reshape(1, N_LABELS))


def kernel(queries, db_weight, db_label, WQ, WK, WV, WO,
           bQ, bK, bV, bO, dense_w, dense_b, out_w, out_b):
    top_v, top_i, table = _run_topk(queries, db_weight, db_label)
    seqs = _sc_gather(table, top_i.reshape(-1))
    M3, VO3, SM3 = _run_prep(WQ, WK, WV, WO, bQ, bK, bV, bO)
    return _run_attn(seqs, queries, top_v, M3, VO3, SM3,
                     dense_w, dense_b, out_w, out_b)
